# baseline (speedup 1.0000x reference)
import jax
import jax.numpy as jnp
from jax.experimental import pallas as pl

H = 32
W = 32
F = 384
B = 16


def _pos_body(row_ref, col_ref, out_ref):
    col_t = col_ref[...].T
    row_t = row_ref[...].T
    x_part = jnp.broadcast_to(col_t[:, None, :], (F, H, W))
    y_part = jnp.broadcast_to(row_t[:, :, None], (F, H, W))
    out_ref[0, :F] = x_part
    out_ref[0, F:] = y_part


def kernel(x, row_embed, col_embed):
    b = x.shape[0]
    out = pl.pallas_call(
        _pos_body,
        grid=(b,),
        in_specs=[
            pl.BlockSpec((H, F), lambda i: (0, 0)),
            pl.BlockSpec((W, F), lambda i: (0, 0)),
        ],
        out_specs=pl.BlockSpec((1, 2 * F, H, W), lambda i: (i, 0, 0, 0)),
        out_shape=jax.ShapeDtypeStruct((b, 2 * F, H, W), jnp.float32),
    )(row_embed, col_embed)
    return out


# speedup vs baseline: 3.3523x; 3.3523x over previous
import jax
import jax.numpy as jnp
from jax import lax
from jax.experimental import pallas as pl
from jax.experimental.pallas import tpu as pltpu

H = 32
W = 32
F = 384
HW = H * W


def _pos_body(row_ref, col_ref, out_ref, scratch):
    pid = pl.program_id(0)

    @pl.when(pid == 0)
    def _():
        col_t = col_ref[...].T
        row_t = row_ref[...].T
        lane = lax.broadcasted_iota(jnp.int32, (W, HW), 1)
        sub = lax.broadcasted_iota(jnp.int32, (W, HW), 0)
        tile_mask = (lane % W == sub).astype(jnp.float32)
        rep_mask = (lane // W == sub).astype(jnp.float32)
        scratch[:F] = jnp.dot(col_t, tile_mask,
                              preferred_element_type=jnp.float32)
        scratch[F:] = jnp.dot(row_t, rep_mask,
                              preferred_element_type=jnp.float32)

    out_ref[0] = scratch[...]


def kernel(x, row_embed, col_embed):
    b = x.shape[0]
    out = pl.pallas_call(
        _pos_body,
        grid=(b,),
        in_specs=[
            pl.BlockSpec((H, F), lambda i: (0, 0)),
            pl.BlockSpec((W, F), lambda i: (0, 0)),
        ],
        out_specs=pl.BlockSpec((1, 2 * F, HW), lambda i: (i, 0, 0)),
        out_shape=jax.ShapeDtypeStruct((b, 2 * F, HW), jnp.float32),
        scratch_shapes=[pltpu.VMEM((2 * F, HW), jnp.float32)],
    )(row_embed, col_embed)
    return out.reshape(b, 2 * F, H, W)
